# Initial kernel scaffold; baseline (speedup 1.0000x reference)
import jax
import jax.numpy as jnp
from jax.experimental import pallas as pl


def kernel(affines):
    raise NotImplementedError("write your pallas kernel here")


# speedup vs baseline: 4.1420x; 4.1420x over previous
import functools
import math

import jax
import jax.numpy as jnp
import numpy as np
from jax.experimental import pallas as pl

_N = 8192
_K = 32
_PED = 64
_HALF = _PED // 2
_BLK = 128
_BIG_I32 = 2**30


def _knn_kernel(pblk_ref, pt_ref, idx_ref):
    i = pl.program_id(0)
    pblk = pblk_ref[...]
    pt = pt_ref[...]
    g = jax.lax.dot_general(
        pblk, pt, (((1,), (0,)), ((), ())),
        preferred_element_type=jnp.float32)
    sqr = jnp.sum(pblk * pblk, axis=1, keepdims=True)
    sqc = jnp.sum(pt * pt, axis=0, keepdims=True)
    d2 = (sqr + sqc) - 2.0 * g
    ci = jax.lax.broadcasted_iota(jnp.int32, (1, _N), 1)
    rows = i * _BLK + jax.lax.broadcasted_iota(jnp.int32, (_BLK, 1), 0)
    inf = jnp.float32(jnp.inf)
    d2 = jnp.where(ci == rows, inf, d2)
    for t in range(_K):
        m = jnp.min(d2, axis=1, keepdims=True)
        j = jnp.min(jnp.where(d2 == m, ci, _BIG_I32), axis=1, keepdims=True)
        idx_ref[:, t:t + 1] = j
        d2 = jnp.where(ci == j, inf, d2)


def _embed_kernel(nx_ref, ny_ref, nz_ref, p3_ref, r9_ref,
                  e32_ref, f2k_ref, m2k_ref, e3_ref, f192_ref, m192_ref,
                  lx_ref, ly_ref, lz_ref, nd_ref, pe_ref):
    p3 = p3_ref[...]
    r9 = r9_ref[...]
    rx = nx_ref[...] - p3[:, 0:1]
    ry = ny_ref[...] - p3[:, 1:2]
    rz = nz_ref[...] - p3[:, 2:3]
    lx = r9[:, 0:1] * rx + r9[:, 3:4] * ry + r9[:, 6:7] * rz
    ly = r9[:, 1:2] * rx + r9[:, 4:5] * ry + r9[:, 7:8] * rz
    lz = r9[:, 2:3] * rx + r9[:, 5:6] * ry + r9[:, 8:9] * rz
    lx_ref[...] = lx
    ly_ref[...] = ly
    lz_ref[...] = lz
    nd = jnp.sqrt(lx * lx + ly * ly + lz * lz)
    nd_exp = jax.lax.dot_general(
        nd, e32_ref[...], (((1,), (0,)), ((), ())),
        preferred_element_type=jnp.float32)
    args = nd_exp * f2k_ref[...]
    nd_ref[...] = jnp.where(m2k_ref[...] > 0, jnp.sin(args), jnp.cos(args))
    p_exp = jax.lax.dot_general(
        p3, e3_ref[...], (((1,), (0,)), ((), ())),
        preferred_element_type=jnp.float32)
    pargs = p_exp * f192_ref[...]
    pe_ref[...] = jnp.where(m192_ref[...] > 0, jnp.sin(pargs), jnp.cos(pargs))


def _knn_topk(positions):
    ppad = jnp.zeros((_N, 128), jnp.float32).at[:, :3].set(positions)
    pt = ppad[:, :128].T
    grid = _N // _BLK
    return pl.pallas_call(
        _knn_kernel,
        grid=(grid,),
        in_specs=[
            pl.BlockSpec((_BLK, 128), lambda i: (i, 0)),
            pl.BlockSpec((128, _N), lambda i: (0, 0)),
        ],
        out_specs=pl.BlockSpec((_BLK, _K), lambda i: (i, 0)),
        out_shape=jax.ShapeDtypeStruct((_N, _K), jnp.int32),
    )(ppad, pt)


def _embed(nx, ny, nz, positions, rot):
    r9 = rot.reshape(_N, 9)
    freqs = jnp.exp(-np.log(10000.0)
                    * jnp.arange(_HALF, dtype=jnp.float32) / _HALF)
    kcol = np.arange(_K * _PED)
    e32 = jnp.asarray(np.eye(_K, dtype=np.float32)[:, kcol // _PED])
    f2k = freqs[jnp.asarray(kcol % _HALF)][None, :]
    m2k = jnp.asarray(((kcol % _PED) < _HALF).astype(np.int32))[None, :]
    icol = np.arange(3 * _PED)
    e3 = jnp.asarray(np.eye(3, dtype=np.float32)[:, icol // _PED])
    f192 = freqs[jnp.asarray(icol % _HALF)][None, :]
    m192 = jnp.asarray(((icol % _PED) < _HALF).astype(np.int32))[None, :]

    grid = _N // _BLK
    row_spec = lambda w: pl.BlockSpec((_BLK, w), lambda i: (i, 0))
    const_spec = lambda h, w: pl.BlockSpec((h, w), lambda i: (0, 0))
    return pl.pallas_call(
        _embed_kernel,
        grid=(grid,),
        in_specs=[
            row_spec(_K), row_spec(_K), row_spec(_K),
            row_spec(3), row_spec(9),
            const_spec(_K, _K * _PED), const_spec(1, _K * _PED),
            const_spec(1, _K * _PED),
            const_spec(3, 3 * _PED), const_spec(1, 3 * _PED),
            const_spec(1, 3 * _PED),
        ],
        out_specs=[row_spec(_K), row_spec(_K), row_spec(_K),
                   row_spec(_K * _PED), row_spec(3 * _PED)],
        out_shape=[
            jax.ShapeDtypeStruct((_N, _K), jnp.float32),
            jax.ShapeDtypeStruct((_N, _K), jnp.float32),
            jax.ShapeDtypeStruct((_N, _K), jnp.float32),
            jax.ShapeDtypeStruct((_N, _K * _PED), jnp.float32),
            jax.ShapeDtypeStruct((_N, 3 * _PED), jnp.float32),
        ],
    )(nx, ny, nz, positions, r9, e32, f2k, m2k, e3, f192, m192)


def kernel(affines):
    positions = affines[:, :3, 3]
    rot = affines[:, :3, :3]
    idx = _knn_topk(positions)
    neigh = positions[idx]
    nx, ny, nz = neigh[..., 0], neigh[..., 1], neigh[..., 2]
    lx, ly, lz, nd, pe = _embed(nx, ny, nz, positions, rot)
    neighbour_positions = jnp.stack([lx, ly, lz], axis=-1)
    neighbour_distances = nd.reshape(_N, _K, _PED)
    targets = jnp.repeat(jnp.arange(_N, dtype=jnp.int32), _K)
    full_edge_index = jnp.stack([idx.reshape(-1), targets], axis=0)
    return (pe, positions, neighbour_positions, neighbour_distances,
            idx, full_edge_index)


# speedup vs baseline: 5.5516x; 1.3403x over previous
import functools
import math

import jax
import jax.numpy as jnp
import numpy as np
from jax import lax
from jax.experimental import pallas as pl
from jax.experimental.pallas import tpu as pltpu
from jax.experimental.pallas import tpu_sc as plsc

_N = 8192
_K = 32
_PED = 64
_HALF = _PED // 2
_BLK = 128
_BIG_I32 = 2**30


def _knn_kernel(pblk_ref, pt_ref, idx_ref):
    i = pl.program_id(0)
    pblk = pblk_ref[...]
    pt = pt_ref[...]
    g = jax.lax.dot_general(
        pblk, pt, (((1,), (0,)), ((), ())),
        preferred_element_type=jnp.float32)
    sqr = jnp.sum(pblk * pblk, axis=1, keepdims=True)
    sqc = jnp.sum(pt * pt, axis=0, keepdims=True)
    d2 = (sqr + sqc) - 2.0 * g
    ci = jax.lax.broadcasted_iota(jnp.int32, (1, _N), 1)
    rows = i * _BLK + jax.lax.broadcasted_iota(jnp.int32, (_BLK, 1), 0)
    inf = jnp.float32(jnp.inf)
    d2 = jnp.where(ci == rows, inf, d2)
    for t in range(_K):
        m = jnp.min(d2, axis=1, keepdims=True)
        j = jnp.min(jnp.where(d2 == m, ci, _BIG_I32), axis=1, keepdims=True)
        idx_ref[:, t:t + 1] = j
        d2 = jnp.where(ci == j, inf, d2)


def _embed_kernel(nx_ref, ny_ref, nz_ref, p3_ref, r9_ref,
                  e32_ref, f2k_ref, m2k_ref, e3_ref, f192_ref, m192_ref,
                  lx_ref, ly_ref, lz_ref, nd_ref, pe_ref):
    p3 = p3_ref[...]
    r9 = r9_ref[...]
    rx = nx_ref[...] - p3[:, 0:1]
    ry = ny_ref[...] - p3[:, 1:2]
    rz = nz_ref[...] - p3[:, 2:3]
    lx = r9[:, 0:1] * rx + r9[:, 3:4] * ry + r9[:, 6:7] * rz
    ly = r9[:, 1:2] * rx + r9[:, 4:5] * ry + r9[:, 7:8] * rz
    lz = r9[:, 2:3] * rx + r9[:, 5:6] * ry + r9[:, 8:9] * rz
    lx_ref[...] = lx
    ly_ref[...] = ly
    lz_ref[...] = lz
    nd = jnp.sqrt(lx * lx + ly * ly + lz * lz)
    nd_exp = jax.lax.dot_general(
        nd, e32_ref[...], (((1,), (0,)), ((), ())),
        preferred_element_type=jnp.float32)
    args = nd_exp * f2k_ref[...]
    nd_ref[...] = jnp.where(m2k_ref[...] > 0, jnp.sin(args), jnp.cos(args))
    p_exp = jax.lax.dot_general(
        p3, e3_ref[...], (((1,), (0,)), ((), ())),
        preferred_element_type=jnp.float32)
    pargs = p_exp * f192_ref[...]
    pe_ref[...] = jnp.where(m192_ref[...] > 0, jnp.sin(pargs), jnp.cos(pargs))


def _knn_topk(ppad):
    pt = ppad[:, :128].T
    grid = _N // _BLK
    return pl.pallas_call(
        _knn_kernel,
        grid=(grid,),
        in_specs=[
            pl.BlockSpec((_BLK, 128), lambda i: (i, 0)),
            pl.BlockSpec((128, _N), lambda i: (0, 0)),
        ],
        out_specs=pl.BlockSpec((_BLK, _K), lambda i: (i, 0)),
        out_shape=jax.ShapeDtypeStruct((_N, _K), jnp.int32),
    )(ppad, pt)


def _embed(nx, ny, nz, positions, rot):
    r9 = rot.reshape(_N, 9)
    freqs = jnp.exp(-np.log(10000.0)
                    * jnp.arange(_HALF, dtype=jnp.float32) / _HALF)
    kcol = np.arange(_K * _PED)
    e32 = jnp.asarray(np.eye(_K, dtype=np.float32)[:, kcol // _PED])
    f2k = freqs[jnp.asarray(kcol % _HALF)][None, :]
    m2k = jnp.asarray(((kcol % _PED) < _HALF).astype(np.int32))[None, :]
    icol = np.arange(3 * _PED)
    e3 = jnp.asarray(np.eye(3, dtype=np.float32)[:, icol // _PED])
    f192 = freqs[jnp.asarray(icol % _HALF)][None, :]
    m192 = jnp.asarray(((icol % _PED) < _HALF).astype(np.int32))[None, :]

    grid = _N // _BLK
    row_spec = lambda w: pl.BlockSpec((_BLK, w), lambda i: (i, 0))
    const_spec = lambda h, w: pl.BlockSpec((h, w), lambda i: (0, 0))
    return pl.pallas_call(
        _embed_kernel,
        grid=(grid,),
        in_specs=[
            row_spec(_K), row_spec(_K), row_spec(_K),
            row_spec(3), row_spec(9),
            const_spec(_K, _K * _PED), const_spec(1, _K * _PED),
            const_spec(1, _K * _PED),
            const_spec(3, 3 * _PED), const_spec(1, 3 * _PED),
            const_spec(1, 3 * _PED),
        ],
        out_specs=[row_spec(_K), row_spec(_K), row_spec(_K),
                   row_spec(_K * _PED), row_spec(3 * _PED)],
        out_shape=[
            jax.ShapeDtypeStruct((_N, _K), jnp.float32),
            jax.ShapeDtypeStruct((_N, _K), jnp.float32),
            jax.ShapeDtypeStruct((_N, _K), jnp.float32),
            jax.ShapeDtypeStruct((_N, _K * _PED), jnp.float32),
            jax.ShapeDtypeStruct((_N, 3 * _PED), jnp.float32),
        ],
    )(nx, ny, nz, positions, r9, e32, f2k, m2k, e3, f192, m192)


def _sc_gather(ppad128, idx_flat):
    info = plsc.get_sparse_core_info()
    nc, ns = info.num_cores, info.num_subcores
    nw = nc * ns
    epw = (_N * _K) // nw
    chunk = 128
    nch = epw // chunk
    idx3 = idx_flat.reshape(nw, nch, chunk)
    mesh = plsc.VectorSubcoreMesh(core_axis_name="c", subcore_axis_name="s")

    @functools.partial(
        pl.kernel, mesh=mesh,
        out_type=jax.ShapeDtypeStruct((nw, epw, 128), jnp.float32),
        scratch_types=[
            pltpu.VMEM((nch, chunk), jnp.int32),
            pltpu.VMEM((chunk, 128), jnp.float32),
            pltpu.SemaphoreType.DMA,
        ],
    )
    def gk(tab_h, idx_h, out_h, vidx, rows, sem):
        wid = lax.axis_index("s") * nc + lax.axis_index("c")
        pltpu.sync_copy(idx_h.at[wid], vidx)
        for c in range(nch):
            pltpu.async_copy(tab_h.at[vidx.at[c]], rows, sem).wait()
            pltpu.sync_copy(rows, out_h.at[wid, pl.ds(c * chunk, chunk)])

    g = gk(ppad128, idx3).reshape(_N * _K, 128)
    return (g[:, 0].reshape(_N, _K), g[:, 1].reshape(_N, _K),
            g[:, 2].reshape(_N, _K))


def kernel(affines):
    positions = affines[:, :3, 3]
    rot = affines[:, :3, :3]
    ppad = jnp.zeros((_N, 128), jnp.float32).at[:, :3].set(positions)
    idx = _knn_topk(ppad)
    nx, ny, nz = _sc_gather(ppad, idx.reshape(-1))
    lx, ly, lz, nd, pe = _embed(nx, ny, nz, positions, rot)
    neighbour_positions = jnp.stack([lx, ly, lz], axis=-1)
    neighbour_distances = nd.reshape(_N, _K, _PED)
    targets = jnp.repeat(jnp.arange(_N, dtype=jnp.int32), _K)
    full_edge_index = jnp.stack([idx.reshape(-1), targets], axis=0)
    return (pe, positions, neighbour_positions, neighbour_distances,
            idx, full_edge_index)


# speedup vs baseline: 5.5910x; 1.0071x over previous
import functools
import math

import jax
import jax.numpy as jnp
import numpy as np
from jax import lax
from jax.experimental import pallas as pl
from jax.experimental.pallas import tpu as pltpu
from jax.experimental.pallas import tpu_sc as plsc

_N = 8192
_K = 32
_PED = 64
_HALF = _PED // 2
_BLK = 128
_BIG_I32 = 2**30


def _knn_kernel(pblk_ref, pt_ref, idx_ref):
    i = pl.program_id(0)
    pblk = pblk_ref[...]
    pt = pt_ref[...]
    g = jax.lax.dot_general(
        pblk, pt, (((1,), (0,)), ((), ())),
        preferred_element_type=jnp.float32)
    sqr = jnp.sum(pblk * pblk, axis=1, keepdims=True)
    sqc = jnp.sum(pt * pt, axis=0, keepdims=True)
    d2 = (sqr + sqc) - 2.0 * g
    ci = jax.lax.broadcasted_iota(jnp.int32, (1, _N), 1)
    rows = i * _BLK + jax.lax.broadcasted_iota(jnp.int32, (_BLK, 1), 0)
    inf = jnp.float32(jnp.inf)
    d2 = jnp.where(ci == rows, inf, d2)
    for t in range(_K):
        m = jnp.min(d2, axis=1, keepdims=True)
        j = jnp.min(jnp.where(d2 == m, ci, _BIG_I32), axis=1, keepdims=True)
        idx_ref[:, t:t + 1] = j
        d2 = jnp.where(ci == j, inf, d2)


def _embed_kernel(nx_ref, ny_ref, nz_ref, p3_ref, r9_ref,
                  e32_ref, f2k_ref, o2k_ref, e3_ref, f192_ref, o192_ref,
                  lx_ref, ly_ref, lz_ref, nd_ref, pe_ref):
    p3 = p3_ref[...]
    r9 = r9_ref[...]
    rx = nx_ref[...] - p3[:, 0:1]
    ry = ny_ref[...] - p3[:, 1:2]
    rz = nz_ref[...] - p3[:, 2:3]
    lx = r9[:, 0:1] * rx + r9[:, 3:4] * ry + r9[:, 6:7] * rz
    ly = r9[:, 1:2] * rx + r9[:, 4:5] * ry + r9[:, 7:8] * rz
    lz = r9[:, 2:3] * rx + r9[:, 5:6] * ry + r9[:, 8:9] * rz
    lx_ref[...] = lx
    ly_ref[...] = ly
    lz_ref[...] = lz
    nd = jnp.sqrt(lx * lx + ly * ly + lz * lz)
    nd_exp = jax.lax.dot_general(
        nd, e32_ref[...], (((1,), (0,)), ((), ())),
        preferred_element_type=jnp.float32)
    nd_ref[...] = jnp.sin(nd_exp * f2k_ref[...] + o2k_ref[...])
    p_exp = jax.lax.dot_general(
        p3, e3_ref[...], (((1,), (0,)), ((), ())),
        preferred_element_type=jnp.float32)
    pe_ref[...] = jnp.sin(p_exp * f192_ref[...] + o192_ref[...])


def _knn_topk(ppad):
    pt = ppad[:, :128].T
    grid = _N // _BLK
    return pl.pallas_call(
        _knn_kernel,
        grid=(grid,),
        in_specs=[
            pl.BlockSpec((_BLK, 128), lambda i: (i, 0)),
            pl.BlockSpec((128, _N), lambda i: (0, 0)),
        ],
        out_specs=pl.BlockSpec((_BLK, _K), lambda i: (i, 0)),
        out_shape=jax.ShapeDtypeStruct((_N, _K), jnp.int32),
    )(ppad, pt)


def _embed(nx, ny, nz, positions, rot):
    r9 = rot.reshape(_N, 9)
    freqs = jnp.exp(-np.log(10000.0)
                    * jnp.arange(_HALF, dtype=jnp.float32) / _HALF)
    half_pi = np.float32(np.pi / 2)
    kcol = np.arange(_K * _PED)
    e32 = jnp.asarray(np.eye(_K, dtype=np.float32)[:, kcol // _PED])
    f2k = freqs[jnp.asarray(kcol % _HALF)][None, :]
    o2k = jnp.asarray(((kcol % _PED) >= _HALF).astype(np.float32)
                      * half_pi)[None, :]
    icol = np.arange(3 * _PED)
    e3 = jnp.asarray(np.eye(3, dtype=np.float32)[:, icol // _PED])
    f192 = freqs[jnp.asarray(icol % _HALF)][None, :]
    o192 = jnp.asarray(((icol % _PED) >= _HALF).astype(np.float32)
                       * half_pi)[None, :]

    grid = _N // _BLK
    row_spec = lambda w: pl.BlockSpec((_BLK, w), lambda i: (i, 0))
    const_spec = lambda h, w: pl.BlockSpec((h, w), lambda i: (0, 0))
    return pl.pallas_call(
        _embed_kernel,
        grid=(grid,),
        in_specs=[
            row_spec(_K), row_spec(_K), row_spec(_K),
            row_spec(3), row_spec(9),
            const_spec(_K, _K * _PED), const_spec(1, _K * _PED),
            const_spec(1, _K * _PED),
            const_spec(3, 3 * _PED), const_spec(1, 3 * _PED),
            const_spec(1, 3 * _PED),
        ],
        out_specs=[row_spec(_K), row_spec(_K), row_spec(_K),
                   row_spec(_K * _PED), row_spec(3 * _PED)],
        out_shape=[
            jax.ShapeDtypeStruct((_N, _K), jnp.float32),
            jax.ShapeDtypeStruct((_N, _K), jnp.float32),
            jax.ShapeDtypeStruct((_N, _K), jnp.float32),
            jax.ShapeDtypeStruct((_N, _K * _PED), jnp.float32),
            jax.ShapeDtypeStruct((_N, 3 * _PED), jnp.float32),
        ],
    )(nx, ny, nz, positions, r9, e32, f2k, o2k, e3, f192, o192)


def _sc_gather(ppad128, idx_flat):
    info = plsc.get_sparse_core_info()
    nc, ns = info.num_cores, info.num_subcores
    nw = nc * ns
    epw = (_N * _K) // nw
    chunk = 128
    nch = epw // chunk
    idx3 = idx_flat.reshape(nw, nch, chunk)
    mesh = plsc.VectorSubcoreMesh(core_axis_name="c", subcore_axis_name="s")

    @functools.partial(
        pl.kernel, mesh=mesh,
        out_type=jax.ShapeDtypeStruct((nw, epw, 128), jnp.float32),
        scratch_types=[
            pltpu.VMEM((nch, chunk), jnp.int32),
            pltpu.VMEM((chunk, 128), jnp.float32),
            pltpu.SemaphoreType.DMA,
        ],
    )
    def gk(tab_h, idx_h, out_h, vidx, rows, sem):
        wid = lax.axis_index("s") * nc + lax.axis_index("c")
        pltpu.sync_copy(idx_h.at[wid], vidx)
        for c in range(nch):
            pltpu.async_copy(tab_h.at[vidx.at[c]], rows, sem).wait()
            pltpu.sync_copy(rows, out_h.at[wid, pl.ds(c * chunk, chunk)])

    g = gk(ppad128, idx3).reshape(_N * _K, 128)
    return (g[:, 0].reshape(_N, _K), g[:, 1].reshape(_N, _K),
            g[:, 2].reshape(_N, _K))


def kernel(affines):
    positions = affines[:, :3, 3]
    rot = affines[:, :3, :3]
    ppad = jnp.zeros((_N, 128), jnp.float32).at[:, :3].set(positions)
    idx = _knn_topk(ppad)
    nx, ny, nz = _sc_gather(ppad, idx.reshape(-1))
    lx, ly, lz, nd, pe = _embed(nx, ny, nz, positions, rot)
    neighbour_positions = jnp.stack([lx, ly, lz], axis=-1)
    neighbour_distances = nd.reshape(_N, _K, _PED)
    targets = jnp.repeat(jnp.arange(_N, dtype=jnp.int32), _K)
    full_edge_index = jnp.stack([idx.reshape(-1), targets], axis=0)
    return (pe, positions, neighbour_positions, neighbour_distances,
            idx, full_edge_index)


# speedup vs baseline: 5.5947x; 1.0007x over previous
import functools
import math

import jax
import jax.numpy as jnp
import numpy as np
from jax import lax
from jax.experimental import pallas as pl
from jax.experimental.pallas import tpu as pltpu
from jax.experimental.pallas import tpu_sc as plsc

_N = 8192
_K = 32
_PED = 64
_HALF = _PED // 2
_BLK = 128
_BIG_I32 = 2**30


def _knn_kernel(pblk_ref, pt_ref, idx_ref):
    i = pl.program_id(0)
    pblk = pblk_ref[...]
    pt = pt_ref[...]
    g = jax.lax.dot_general(
        pblk, pt, (((1,), (0,)), ((), ())),
        preferred_element_type=jnp.float32)
    sqr = jnp.sum(pblk * pblk, axis=1, keepdims=True)
    sqc = jnp.sum(pt * pt, axis=0, keepdims=True)
    d2 = (sqr + sqc) - 2.0 * g
    ci = jax.lax.broadcasted_iota(jnp.int32, (1, _N), 1)
    rows = i * _BLK + jax.lax.broadcasted_iota(jnp.int32, (_BLK, 1), 0)
    inf = jnp.float32(jnp.inf)
    d2 = jnp.where(ci == rows, inf, d2)
    for t in range(_K):
        m = jnp.min(d2, axis=1, keepdims=True)
        j = jnp.min(jnp.where(d2 == m, ci, _BIG_I32), axis=1, keepdims=True)
        idx_ref[:, t:t + 1] = j
        d2 = jnp.where(ci == j, inf, d2)


def _embed_kernel(nx_ref, ny_ref, nz_ref, p3_ref, r9_ref,
                  e32_ref, f2k_ref, o2k_ref, e3_ref, f192_ref, o192_ref,
                  lx_ref, ly_ref, lz_ref, nd_ref, pe_ref):
    p3 = p3_ref[...]
    r9 = r9_ref[...]
    rx = nx_ref[...] - p3[:, 0:1]
    ry = ny_ref[...] - p3[:, 1:2]
    rz = nz_ref[...] - p3[:, 2:3]
    lx = r9[:, 0:1] * rx + r9[:, 3:4] * ry + r9[:, 6:7] * rz
    ly = r9[:, 1:2] * rx + r9[:, 4:5] * ry + r9[:, 7:8] * rz
    lz = r9[:, 2:3] * rx + r9[:, 5:6] * ry + r9[:, 8:9] * rz
    lx_ref[...] = lx
    ly_ref[...] = ly
    lz_ref[...] = lz
    nd = jnp.sqrt(lx * lx + ly * ly + lz * lz)
    nd_exp = jax.lax.dot_general(
        nd, e32_ref[...], (((1,), (0,)), ((), ())),
        preferred_element_type=jnp.float32)
    nd_ref[...] = jnp.sin(nd_exp * f2k_ref[...] + o2k_ref[...])
    p_exp = jax.lax.dot_general(
        p3, e3_ref[...], (((1,), (0,)), ((), ())),
        preferred_element_type=jnp.float32)
    pe_ref[...] = jnp.sin(p_exp * f192_ref[...] + o192_ref[...])


def _knn_topk(ppad):
    pt = ppad[:, :128].T
    grid = _N // _BLK
    return pl.pallas_call(
        _knn_kernel,
        grid=(grid,),
        in_specs=[
            pl.BlockSpec((_BLK, 128), lambda i: (i, 0)),
            pl.BlockSpec((128, _N), lambda i: (0, 0)),
        ],
        out_specs=pl.BlockSpec((_BLK, _K), lambda i: (i, 0)),
        out_shape=jax.ShapeDtypeStruct((_N, _K), jnp.int32),
        compiler_params=pltpu.CompilerParams(
            dimension_semantics=("parallel",)),
    )(ppad, pt)


def _embed(nx, ny, nz, positions, rot):
    r9 = rot.reshape(_N, 9)
    freqs = jnp.exp(-np.log(10000.0)
                    * jnp.arange(_HALF, dtype=jnp.float32) / _HALF)
    half_pi = np.float32(np.pi / 2)
    kcol = np.arange(_K * _PED)
    e32 = jnp.asarray(np.eye(_K, dtype=np.float32)[:, kcol // _PED])
    f2k = freqs[jnp.asarray(kcol % _HALF)][None, :]
    o2k = jnp.asarray(((kcol % _PED) >= _HALF).astype(np.float32)
                      * half_pi)[None, :]
    icol = np.arange(3 * _PED)
    e3 = jnp.asarray(np.eye(3, dtype=np.float32)[:, icol // _PED])
    f192 = freqs[jnp.asarray(icol % _HALF)][None, :]
    o192 = jnp.asarray(((icol % _PED) >= _HALF).astype(np.float32)
                       * half_pi)[None, :]

    grid = _N // _BLK
    row_spec = lambda w: pl.BlockSpec((_BLK, w), lambda i: (i, 0))
    const_spec = lambda h, w: pl.BlockSpec((h, w), lambda i: (0, 0))
    return pl.pallas_call(
        _embed_kernel,
        grid=(grid,),
        in_specs=[
            row_spec(_K), row_spec(_K), row_spec(_K),
            row_spec(3), row_spec(9),
            const_spec(_K, _K * _PED), const_spec(1, _K * _PED),
            const_spec(1, _K * _PED),
            const_spec(3, 3 * _PED), const_spec(1, 3 * _PED),
            const_spec(1, 3 * _PED),
        ],
        out_specs=[row_spec(_K), row_spec(_K), row_spec(_K),
                   row_spec(_K * _PED), row_spec(3 * _PED)],
        out_shape=[
            jax.ShapeDtypeStruct((_N, _K), jnp.float32),
            jax.ShapeDtypeStruct((_N, _K), jnp.float32),
            jax.ShapeDtypeStruct((_N, _K), jnp.float32),
            jax.ShapeDtypeStruct((_N, _K * _PED), jnp.float32),
            jax.ShapeDtypeStruct((_N, 3 * _PED), jnp.float32),
        ],
        compiler_params=pltpu.CompilerParams(
            dimension_semantics=("parallel",)),
    )(nx, ny, nz, positions, r9, e32, f2k, o2k, e3, f192, o192)


def _sc_gather(ppad128, idx_flat):
    info = plsc.get_sparse_core_info()
    nc, ns = info.num_cores, info.num_subcores
    nw = nc * ns
    epw = (_N * _K) // nw
    chunk = 128
    nch = epw // chunk
    idx3 = idx_flat.reshape(nw, nch, chunk)
    mesh = plsc.VectorSubcoreMesh(core_axis_name="c", subcore_axis_name="s")

    @functools.partial(
        pl.kernel, mesh=mesh,
        out_type=jax.ShapeDtypeStruct((nw, epw, 128), jnp.float32),
        scratch_types=[
            pltpu.VMEM((nch, chunk), jnp.int32),
            pltpu.VMEM((chunk, 128), jnp.float32),
            pltpu.SemaphoreType.DMA,
        ],
    )
    def gk(tab_h, idx_h, out_h, vidx, rows, sem):
        wid = lax.axis_index("s") * nc + lax.axis_index("c")
        pltpu.sync_copy(idx_h.at[wid], vidx)
        for c in range(nch):
            pltpu.async_copy(tab_h.at[vidx.at[c]], rows, sem).wait()
            pltpu.sync_copy(rows, out_h.at[wid, pl.ds(c * chunk, chunk)])

    g = gk(ppad128, idx3).reshape(_N * _K, 128)
    return (g[:, 0].reshape(_N, _K), g[:, 1].reshape(_N, _K),
            g[:, 2].reshape(_N, _K))


def kernel(affines):
    positions = affines[:, :3, 3]
    rot = affines[:, :3, :3]
    ppad = jnp.zeros((_N, 128), jnp.float32).at[:, :3].set(positions)
    idx = _knn_topk(ppad)
    nx, ny, nz = _sc_gather(ppad, idx.reshape(-1))
    lx, ly, lz, nd, pe = _embed(nx, ny, nz, positions, rot)
    neighbour_positions = jnp.stack([lx, ly, lz], axis=-1)
    neighbour_distances = nd.reshape(_N, _K, _PED)
    targets = jnp.repeat(jnp.arange(_N, dtype=jnp.int32), _K)
    full_edge_index = jnp.stack([idx.reshape(-1), targets], axis=0)
    return (pe, positions, neighbour_positions, neighbour_distances,
            idx, full_edge_index)
